# Initial kernel scaffold; baseline (speedup 1.0000x reference)
import jax
import jax.numpy as jnp
from jax.experimental import pallas as pl


def kernel(x, edge_index, W1_self, W1_neigh, b1, W2_self, W2_neigh, b2):
    raise NotImplementedError("write your pallas kernel here")


# speedup vs baseline: 4.6497x; 4.6497x over previous
import functools

import jax
import jax.numpy as jnp
from jax import lax
from jax.experimental import pallas as pl
from jax.experimental.pallas import tpu as pltpu
from jax.experimental.pallas import tpu_sc as plsc

NC = 2
NS = 16
NW = NC * NS
CHUNK = 128
LANES = 16


def _sc_agg_builder(n_nodes, d, e_pad, with_deg):
    chunks_per_tile = e_pad // (NW * CHUNK)
    n_pad = ((n_nodes + 1 + NS * CHUNK - 1) // (NS * CHUNK)) * (NS * CHUNK)
    rows_per_tile = n_pad // NS
    zrows = rows_per_tile // CHUNK

    mesh = plsc.VectorSubcoreMesh(core_axis_name="c", subcore_axis_name="s",
                                  num_cores=NC, num_subcores=NS)

    out_type = [jax.ShapeDtypeStruct((NC, n_pad, d), jnp.float32)]
    scratch = [
        pltpu.VMEM_SHARED((n_pad, d), jnp.float32),
        pltpu.VMEM((CHUNK,), jnp.int32),
        pltpu.VMEM((CHUNK,), jnp.int32),
        pltpu.VMEM((CHUNK, d), jnp.float32),
        pltpu.SemaphoreType.DMA,
    ]
    if with_deg:
        out_type.append(jax.ShapeDtypeStruct((NC, NS, n_pad), jnp.float32))
        scratch.append(pltpu.VMEM((n_pad,), jnp.float32))

    def body(table_hbm, src_hbm, dst_hbm, agg_hbm, *rest):
        if with_deg:
            deg_hbm, agg_sh, src_v, dst_v, rows_v, sem, deg_v = rest
        else:
            agg_sh, src_v, dst_v, rows_v, sem = rest
        cid = lax.axis_index("c")
        sid = lax.axis_index("s")
        wid = cid * NS + sid

        zeros16 = jnp.zeros((LANES,), jnp.float32)

        def zb_loop(i, carry):
            for k in range(d // LANES):
                rows_v[i, pl.ds(k * LANES, LANES)] = zeros16
            return carry
        lax.fori_loop(0, CHUNK, zb_loop, 0)

        if with_deg:
            def zd_loop(i, carry):
                deg_v[pl.ds(i * LANES, LANES)] = zeros16
                return carry
            lax.fori_loop(0, n_pad // LANES, zd_loop, 0)

        rbase = sid * rows_per_tile
        for b in range(zrows):
            pltpu.sync_copy(rows_v, agg_sh.at[pl.ds(rbase + b * CHUNK, CHUNK)])

        plsc.subcore_barrier()

        ebase = wid * (chunks_per_tile * CHUNK)

        def edge_loop(j, carry):
            b = ebase + j * CHUNK
            pltpu.sync_copy(src_hbm.at[pl.ds(b, CHUNK)], src_v)
            pltpu.sync_copy(dst_hbm.at[pl.ds(b, CHUNK)], dst_v)
            pltpu.async_copy(table_hbm.at[src_v], rows_v, sem).wait()
            pltpu.sync_copy(rows_v, agg_sh.at[dst_v], add=True)
            if with_deg:
                for k in range(CHUNK // LANES):
                    d16 = dst_v[pl.ds(k * LANES, LANES)]
                    cnt, last = plsc.scan_count(d16)
                    plsc.addupdate_scatter(deg_v, [d16],
                                           cnt.astype(jnp.float32), mask=last)
            return carry
        lax.fori_loop(0, chunks_per_tile, edge_loop, 0)

        plsc.subcore_barrier()

        pltpu.sync_copy(agg_sh.at[pl.ds(rbase, rows_per_tile)],
                        agg_hbm.at[cid, pl.ds(rbase, rows_per_tile)])
        if with_deg:
            pltpu.sync_copy(deg_v, deg_hbm.at[cid, sid])

    kern = pl.kernel(
        body,
        out_type=out_type if with_deg else out_type[0],
        mesh=mesh,
        scratch_types=scratch,
        compiler_params=pltpu.CompilerParams(needs_layout_passes=False),
    )
    return kern, n_pad


def _tc_layer_builder(n, d, n_pad, relu, block_rows):
    grid = n_pad // block_rows

    def body(x_ref, agg_ref, deg_ref, ws_ref, wn_ref, b_ref, o_ref):
        agg = agg_ref[0] + agg_ref[1]
        deg = jnp.sum(deg_ref[...], axis=(0, 1))
        inv = 1.0 / jnp.maximum(deg, 1.0)
        hn = agg * inv[:, None]
        y = (jnp.dot(x_ref[...], ws_ref[...], preferred_element_type=jnp.float32)
             + jnp.dot(hn, wn_ref[...], preferred_element_type=jnp.float32)
             + b_ref[...])
        if relu:
            y = jnp.maximum(y, 0.0)
        o_ref[...] = y

    return pl.pallas_call(
        body,
        grid=(grid,),
        in_specs=[
            pl.BlockSpec((block_rows, d), lambda i: (i, 0)),
            pl.BlockSpec((NC, block_rows, d), lambda i: (0, i, 0)),
            pl.BlockSpec((NC, NS, block_rows), lambda i: (0, 0, i)),
            pl.BlockSpec((d, d), lambda i: (0, 0)),
            pl.BlockSpec((d, d), lambda i: (0, 0)),
            pl.BlockSpec((1, d), lambda i: (0, 0)),
        ],
        out_specs=pl.BlockSpec((block_rows, d), lambda i: (i, 0)),
        out_shape=jax.ShapeDtypeStruct((n, d), jnp.float32),
    )


@functools.cache
def _build(n_nodes, d, n_edges):
    chunks_per_tile = -(-n_edges // (NW * CHUNK))
    e_pad = NW * chunks_per_tile * CHUNK
    sc1, n_pad = _sc_agg_builder(n_nodes, d, e_pad, with_deg=True)
    sc2, _ = _sc_agg_builder(n_nodes, d, e_pad, with_deg=False)
    tc1 = _tc_layer_builder(n_nodes, d, n_pad, relu=True, block_rows=1024)
    tc2 = _tc_layer_builder(n_nodes, d, n_pad, relu=False, block_rows=1024)
    return sc1, sc2, tc1, tc2, e_pad


def kernel(x, edge_index, W1_self, W1_neigh, b1, W2_self, W2_neigh, b2):
    n, d = x.shape
    e = edge_index.shape[1]
    sc1, sc2, tc1, tc2, e_pad = _build(n, d, e)

    src = edge_index[0].astype(jnp.int32)
    dst = edge_index[1].astype(jnp.int32)
    pad = e_pad - e
    if pad:
        src = jnp.concatenate([src, jnp.zeros((pad,), jnp.int32)])
        dst = jnp.concatenate([dst, jnp.full((pad,), n, jnp.int32)])

    agg1, deg = sc1(x, src, dst)
    h = tc1(x, agg1, deg, W1_self, W1_neigh, b1.reshape(1, d))
    agg2 = sc2(h, src, dst)
    out = tc2(h, agg2, deg, W2_self, W2_neigh, b2.reshape(1, d))
    return out
